# baseline (speedup 1.0000x reference)
import jax
import jax.numpy as jnp
from jax.experimental import pallas as pl
from jax.experimental.pallas import tpu as pltpu

_D = 1024
_T = 50
_BT = 32
_BM = 512


def _proj_body(cond_ref, w_ref, b_ref, ce_ref):
    ce_ref[...] = jax.lax.dot_general(
        cond_ref[...], w_ref[...],
        dimension_numbers=(((1,), (1,)), ((), ())),
        preferred_element_type=jnp.float32,
    ) + b_ref[...]


def _ln_body(ce_ref, te_ref, g_ref, be_ref, out_ref):
    ce = ce_ref[...]
    te = te_ref[...]
    ones = jnp.ones((1, _D), jnp.float32)
    aug = jnp.concatenate([te, ones], axis=0)
    s = jax.lax.dot_general(
        aug, ce, dimension_numbers=(((1,), (1,)), ((), ())),
        preferred_element_type=jnp.float32)
    cross = s[:_T, :]
    s_ce = s[_T:_T + 1, :]
    ssq_ce = jax.lax.dot_general(
        ones, ce * ce, dimension_numbers=(((1,), (1,)), ((), ())),
        preferred_element_type=jnp.float32)
    s_te = jnp.sum(te, axis=1, keepdims=True)
    ssq_te = jnp.sum(te * te, axis=1, keepdims=True)
    inv_d = jnp.float32(1.0 / _D)
    mean = (s_te + s_ce) * inv_d
    e2 = (ssq_te + ssq_ce + 2.0 * cross) * inv_d
    rs = jax.lax.rsqrt(e2 - mean * mean + 1e-5)
    q = ce[None, :, :] + te[:, None, :]
    out_ref[...] = ((q - mean[:, :, None]) * rs[:, :, None]
                    * g_ref[...][None, :, :] + be_ref[...][None, :, :])


def kernel(T_future, batch_size, cond, time_embedding, W, b, gamma, beta):
    B = cond.shape[0]
    residual = (T_future - _T) + (batch_size - B)
    start = (1 + residual).astype(jnp.int32)
    te50 = jax.lax.dynamic_slice(time_embedding, (start, 0), (_T, _D))
    b2 = b.reshape((1, _D))
    g2 = gamma.reshape((1, _D))
    be2 = beta.reshape((1, _D))

    ce = pl.pallas_call(
        _proj_body,
        grid=(B // _BM,),
        in_specs=[
            pl.BlockSpec((_BM, cond.shape[1]), lambda i: (i, 0)),
            pl.BlockSpec((_D, cond.shape[1]), lambda i: (0, 0)),
            pl.BlockSpec((1, _D), lambda i: (0, 0)),
        ],
        out_specs=pl.BlockSpec((_BM, _D), lambda i: (i, 0)),
        out_shape=jax.ShapeDtypeStruct((B, _D), jnp.float32),
        compiler_params=pltpu.CompilerParams(
            dimension_semantics=("parallel",)),
    )(cond, W, b2)

    out = pl.pallas_call(
        _ln_body,
        grid=(B // _BT,),
        in_specs=[
            pl.BlockSpec((_BT, _D), lambda i: (i, 0)),
            pl.BlockSpec((_T, _D), lambda i: (0, 0)),
            pl.BlockSpec((1, _D), lambda i: (0, 0)),
            pl.BlockSpec((1, _D), lambda i: (0, 0)),
        ],
        out_specs=pl.BlockSpec((_T, _BT, _D), lambda i: (0, i, 0)),
        out_shape=jax.ShapeDtypeStruct((_T, B, _D), jnp.float32),
        compiler_params=pltpu.CompilerParams(
            dimension_semantics=("parallel",)),
    )(ce, te50, g2, be2)
    return jnp.transpose(out, (1, 0, 2))


# speedup vs baseline: 1.1142x; 1.1142x over previous
import jax
import jax.numpy as jnp
from jax.experimental import pallas as pl
from jax.experimental.pallas import tpu as pltpu

_D = 1024
_T = 50
_BT = 64
_BM = 512


def _proj_body(cond_ref, w_ref, b_ref, ce_ref):
    ce_ref[...] = jax.lax.dot_general(
        cond_ref[...], w_ref[...],
        dimension_numbers=(((1,), (1,)), ((), ())),
        preferred_element_type=jnp.float32,
    ) + b_ref[...]


def _ln_body(ce_ref, te_ref, g_ref, be_ref, out_ref):
    ce = ce_ref[...]
    te = te_ref[...]
    ones = jnp.ones((1, _D), jnp.float32)
    aug = jnp.concatenate([te, ones], axis=0)
    s = jax.lax.dot_general(
        aug, ce, dimension_numbers=(((1,), (1,)), ((), ())),
        preferred_element_type=jnp.float32)
    cross = s[:_T, :]
    s_ce = s[_T:_T + 1, :]
    ssq_ce = jax.lax.dot_general(
        ones, ce * ce, dimension_numbers=(((1,), (1,)), ((), ())),
        preferred_element_type=jnp.float32)
    s_te = jnp.sum(te, axis=1, keepdims=True)
    ssq_te = jnp.sum(te * te, axis=1, keepdims=True)
    inv_d = jnp.float32(1.0 / _D)
    mean = (s_te + s_ce) * inv_d
    e2 = (ssq_te + ssq_ce + 2.0 * cross) * inv_d
    rs = jax.lax.rsqrt(e2 - mean * mean + 1e-5)
    q = ce[None, :, :] + te[:, None, :]
    out_ref[...] = ((q - mean[:, :, None]) * rs[:, :, None]
                    * g_ref[...][None, :, :] + be_ref[...][None, :, :])


def kernel(T_future, batch_size, cond, time_embedding, W, b, gamma, beta):
    B = cond.shape[0]
    residual = (T_future - _T) + (batch_size - B)
    start = (1 + residual).astype(jnp.int32)
    te50 = jax.lax.dynamic_slice(time_embedding, (start, 0), (_T, _D))
    b2 = b.reshape((1, _D))
    g2 = gamma.reshape((1, _D))
    be2 = beta.reshape((1, _D))

    ce = pl.pallas_call(
        _proj_body,
        grid=(B // _BM,),
        in_specs=[
            pl.BlockSpec((_BM, cond.shape[1]), lambda i: (i, 0)),
            pl.BlockSpec((_D, cond.shape[1]), lambda i: (0, 0)),
            pl.BlockSpec((1, _D), lambda i: (0, 0)),
        ],
        out_specs=pl.BlockSpec((_BM, _D), lambda i: (i, 0)),
        out_shape=jax.ShapeDtypeStruct((B, _D), jnp.float32),
        compiler_params=pltpu.CompilerParams(
            dimension_semantics=("parallel",)),
    )(cond, W, b2)

    out = pl.pallas_call(
        _ln_body,
        grid=(B // _BT,),
        in_specs=[
            pl.BlockSpec((_BT, _D), lambda i: (i, 0)),
            pl.BlockSpec((_T, _D), lambda i: (0, 0)),
            pl.BlockSpec((1, _D), lambda i: (0, 0)),
            pl.BlockSpec((1, _D), lambda i: (0, 0)),
        ],
        out_specs=pl.BlockSpec((_T, _BT, _D), lambda i: (0, i, 0)),
        out_shape=jax.ShapeDtypeStruct((_T, B, _D), jnp.float32),
        compiler_params=pltpu.CompilerParams(
            dimension_semantics=("parallel",)),
    )(ce, te50, g2, be2)
    return jnp.transpose(out, (1, 0, 2))


# speedup vs baseline: 1.1447x; 1.0274x over previous
import jax
import jax.numpy as jnp
from jax.experimental import pallas as pl
from jax.experimental.pallas import tpu as pltpu

_D = 1024
_T = 50
_BT = 64
_BM = 256
_NM = 1024 // _BM


def _body(cond_ref, w_ref, b_ref, te_ref, g_ref, be_ref, out_ref, ce_ref):
    s = pl.program_id(0)

    @pl.when(s < _NM)
    def _():
        part = jax.lax.dot_general(
            cond_ref[...], w_ref[...],
            dimension_numbers=(((1,), (1,)), ((), ())),
            preferred_element_type=jnp.float32,
        ) + b_ref[...]
        ce_ref[pl.ds(pl.multiple_of(s * _BM, _BM), _BM), :] = part

    @pl.when(s >= _NM)
    def _():
        i = s - _NM
        ce = ce_ref[pl.ds(pl.multiple_of(i * _BT, _BT), _BT), :]
        te = te_ref[...]
        ones = jnp.ones((1, _D), jnp.float32)
        aug = jnp.concatenate([te, ones], axis=0)
        st = jax.lax.dot_general(
            aug, ce, dimension_numbers=(((1,), (1,)), ((), ())),
            preferred_element_type=jnp.float32)
        cross = st[:_T, :]
        s_ce = st[_T:_T + 1, :]
        ssq_ce = jax.lax.dot_general(
            ones, ce * ce, dimension_numbers=(((1,), (1,)), ((), ())),
            preferred_element_type=jnp.float32)
        s_te = jnp.sum(te, axis=1, keepdims=True)
        ssq_te = jnp.sum(te * te, axis=1, keepdims=True)
        inv_d = jnp.float32(1.0 / _D)
        mean = (s_te + s_ce) * inv_d
        e2 = (ssq_te + ssq_ce + 2.0 * cross) * inv_d
        rs = jax.lax.rsqrt(e2 - mean * mean + 1e-5)
        q = ce[None, :, :] + te[:, None, :]
        out_ref[...] = ((q - mean[:, :, None]) * rs[:, :, None]
                        * g_ref[...][None, :, :] + be_ref[...][None, :, :])


def kernel(T_future, batch_size, cond, time_embedding, W, b, gamma, beta):
    B = cond.shape[0]
    residual = (T_future - _T) + (batch_size - B)
    start = (1 + residual).astype(jnp.int32)
    te50 = jax.lax.dynamic_slice(time_embedding, (start, 0), (_T, _D))
    b2 = b.reshape((1, _D))
    g2 = gamma.reshape((1, _D))
    be2 = beta.reshape((1, _D))
    nm = _NM
    nb = B // _BT

    out = pl.pallas_call(
        _body,
        grid=(nm + nb,),
        in_specs=[
            pl.BlockSpec((_BM, cond.shape[1]),
                         lambda s: (jnp.minimum(s, nm - 1), 0)),
            pl.BlockSpec((_D, cond.shape[1]), lambda s: (0, 0)),
            pl.BlockSpec((1, _D), lambda s: (0, 0)),
            pl.BlockSpec((_T, _D), lambda s: (0, 0)),
            pl.BlockSpec((1, _D), lambda s: (0, 0)),
            pl.BlockSpec((1, _D), lambda s: (0, 0)),
        ],
        out_specs=pl.BlockSpec(
            (_T, _BT, _D), lambda s: (0, jnp.maximum(s - nm, 0), 0)),
        out_shape=jax.ShapeDtypeStruct((_T, B, _D), jnp.float32),
        scratch_shapes=[pltpu.VMEM((B, _D), jnp.float32)],
        compiler_params=pltpu.CompilerParams(
            dimension_semantics=("arbitrary",)),
    )(cond, W, b2, te50, g2, be2)
    return jnp.transpose(out, (1, 0, 2))


# speedup vs baseline: 1.1493x; 1.0040x over previous
import jax
import jax.numpy as jnp
from jax.experimental import pallas as pl
from jax.experimental.pallas import tpu as pltpu

_D = 1024
_T = 50
_BT = 64
_BM = 512
_NM = 1024 // _BM


def _body(cond_ref, w_ref, b_ref, te_ref, g_ref, be_ref, out_ref, ce_ref):
    s = pl.program_id(0)

    @pl.when(s < _NM)
    def _():
        part = jax.lax.dot_general(
            cond_ref[...], w_ref[...],
            dimension_numbers=(((1,), (1,)), ((), ())),
            preferred_element_type=jnp.float32,
        ) + b_ref[...]
        ce_ref[pl.ds(pl.multiple_of(s * _BM, _BM), _BM), :] = part

    @pl.when(s >= _NM)
    def _():
        i = s - _NM
        ce = ce_ref[pl.ds(pl.multiple_of(i * _BT, _BT), _BT), :]
        te = te_ref[...]
        ones = jnp.ones((1, _D), jnp.float32)
        aug = jnp.concatenate([te, ones], axis=0)
        st = jax.lax.dot_general(
            aug, ce, dimension_numbers=(((1,), (1,)), ((), ())),
            preferred_element_type=jnp.float32)
        cross = st[:_T, :]
        s_ce = st[_T:_T + 1, :]
        ssq_ce = jax.lax.dot_general(
            ones, ce * ce, dimension_numbers=(((1,), (1,)), ((), ())),
            preferred_element_type=jnp.float32)
        s_te = jnp.sum(te, axis=1, keepdims=True)
        ssq_te = jnp.sum(te * te, axis=1, keepdims=True)
        inv_d = jnp.float32(1.0 / _D)
        mean = (s_te + s_ce) * inv_d
        e2 = (ssq_te + ssq_ce + 2.0 * cross) * inv_d
        rs = jax.lax.rsqrt(e2 - mean * mean + 1e-5)
        q = ce[None, :, :] + te[:, None, :]
        out_ref[...] = ((q - mean[:, :, None]) * rs[:, :, None]
                        * g_ref[...][None, :, :] + be_ref[...][None, :, :])


def kernel(T_future, batch_size, cond, time_embedding, W, b, gamma, beta):
    B = cond.shape[0]
    residual = (T_future - _T) + (batch_size - B)
    start = (1 + residual).astype(jnp.int32)
    te50 = jax.lax.dynamic_slice(time_embedding, (start, 0), (_T, _D))
    b2 = b.reshape((1, _D))
    g2 = gamma.reshape((1, _D))
    be2 = beta.reshape((1, _D))
    nm = _NM
    nb = B // _BT

    out = pl.pallas_call(
        _body,
        grid=(nm + nb,),
        in_specs=[
            pl.BlockSpec((_BM, cond.shape[1]),
                         lambda s: (jnp.minimum(s, nm - 1), 0)),
            pl.BlockSpec((_D, cond.shape[1]), lambda s: (0, 0)),
            pl.BlockSpec((1, _D), lambda s: (0, 0)),
            pl.BlockSpec((_T, _D), lambda s: (0, 0)),
            pl.BlockSpec((1, _D), lambda s: (0, 0)),
            pl.BlockSpec((1, _D), lambda s: (0, 0)),
        ],
        out_specs=pl.BlockSpec(
            (_T, _BT, _D), lambda s: (0, jnp.maximum(s - nm, 0), 0)),
        out_shape=jax.ShapeDtypeStruct((_T, B, _D), jnp.float32),
        scratch_shapes=[pltpu.VMEM((B, _D), jnp.float32)],
        compiler_params=pltpu.CompilerParams(
            dimension_semantics=("arbitrary",)),
    )(cond, W, b2, te50, g2, be2)
    return jnp.transpose(out, (1, 0, 2))
